# Initial kernel scaffold; baseline (speedup 1.0000x reference)
import jax
import jax.numpy as jnp
from jax.experimental import pallas as pl


def kernel(features):
    raise NotImplementedError("write your pallas kernel here")


# speedup vs baseline: 120.7542x; 120.7542x over previous
import jax
import jax.numpy as jnp
from jax.experimental import pallas as pl

_TIMESTEPS = 32
_WINDOW = 10


def _body(x_ref, o_ref):
    x = x_ref[0]
    times = (jax.nn.sigmoid(x) * _WINDOW).astype(jnp.int32)
    t_iota = jax.lax.broadcasted_iota(jnp.int32, (_TIMESTEPS, 1, 1), 0)
    o_ref[0] = (times[None, :, :] == t_iota).astype(jnp.float32)


def kernel(features):
    B, S, F = features.shape
    return pl.pallas_call(
        _body,
        grid=(B,),
        in_specs=[pl.BlockSpec((1, S, F), lambda b: (b, 0, 0))],
        out_specs=pl.BlockSpec((1, _TIMESTEPS, S, F), lambda b: (b, 0, 0, 0)),
        out_shape=jax.ShapeDtypeStruct((B, _TIMESTEPS, S, F), jnp.float32),
    )(features)
